# baseline (speedup 1.0000x reference)
import jax
import jax.numpy as jnp
from jax.experimental import pallas as pl
from jax.experimental.pallas import tpu as pltpu

_PAR = pltpu.CompilerParams(dimension_semantics=("parallel",))

_N = 4096
_D = 512
_H = 256
_K = 15
_BLK = 256


def _proj_kernel(feat_ref, w_ref, b_ref, z_ref):
    z = jnp.dot(feat_ref[...], w_ref[...],
                preferred_element_type=jnp.float32) + b_ref[...]
    norm = jnp.sqrt(jnp.sum(z * z, axis=-1, keepdims=True))
    z_ref[...] = z / jnp.maximum(norm, 1e-12)


def _sim_kernel(zb_ref, z_ref, out_ref):
    i = pl.program_id(0)
    s = jax.lax.dot_general(zb_ref[...], z_ref[...], (((1,), (1,)), ((), ())),
                            preferred_element_type=jnp.float32)
    m = jnp.max(s, axis=-1, keepdims=True)
    for _ in range(_K - 1):
        m = jnp.max(jnp.where(s < m, s, -jnp.inf), axis=-1, keepdims=True)
    rows = jax.lax.broadcasted_iota(jnp.int32, (_BLK, _N), 0) + i * _BLK
    cols = jax.lax.broadcasted_iota(jnp.int32, (_BLK, _N), 1)
    keep = (s >= m) & (rows != cols)
    out_ref[...] = jnp.where(keep, s, 0.0)


def kernel(feat, W, b):
    z = pl.pallas_call(
        _proj_kernel,
        grid=(_N // _BLK,),
        in_specs=[pl.BlockSpec((_BLK, _D), lambda i: (i, 0)),
                  pl.BlockSpec((_D, _H), lambda i: (0, 0)),
                  pl.BlockSpec((1, _H), lambda i: (0, 0))],
        out_specs=pl.BlockSpec((_BLK, _H), lambda i: (i, 0)),
        out_shape=jax.ShapeDtypeStruct((_N, _H), jnp.float32),
        compiler_params=_PAR,
    )(feat, W, b.reshape(1, _H))
    out = pl.pallas_call(
        _sim_kernel,
        grid=(_N // _BLK,),
        in_specs=[pl.BlockSpec((_BLK, _H), lambda i: (i, 0)),
                  pl.BlockSpec((_N, _H), lambda i: (0, 0))],
        out_specs=pl.BlockSpec((_BLK, _N), lambda i: (i, 0)),
        out_shape=jax.ShapeDtypeStruct((_N, _N), jnp.float32),
        compiler_params=_PAR,
    )(z, z)
    return out


# speedup vs baseline: 1.6772x; 1.6772x over previous
import jax
import jax.numpy as jnp
from jax.experimental import pallas as pl
from jax.experimental.pallas import tpu as pltpu

_PAR = pltpu.CompilerParams(dimension_semantics=("parallel",))

_N = 4096
_D = 512
_H = 256
_K = 15
_BLK = 256


def _proj_kernel(feat_ref, w_ref, b_ref, z_ref):
    z = jnp.dot(feat_ref[...], w_ref[...],
                preferred_element_type=jnp.float32) + b_ref[...]
    norm = jnp.sqrt(jnp.sum(z * z, axis=-1, keepdims=True))
    z_ref[...] = z / jnp.maximum(norm, 1e-12)


def _sim_kernel(zb_ref, z_ref, out_ref):
    i = pl.program_id(0)
    s = jax.lax.dot_general(zb_ref[...], z_ref[...], (((1,), (1,)), ((), ())),
                            preferred_element_type=jnp.float32)
    neg = jnp.full((_BLK, 128), -jnp.inf, dtype=jnp.float32)
    m1, m2, m3, m4 = neg, neg, neg, neg
    for c in range(_N // 128):
        v = s[:, c * 128:(c + 1) * 128]
        r = jnp.minimum(m1, v)
        m1 = jnp.maximum(m1, v)
        r, m2 = jnp.minimum(m2, r), jnp.maximum(m2, r)
        r, m3 = jnp.minimum(m3, r), jnp.maximum(m3, r)
        m4 = jnp.maximum(m4, r)
    cand = jnp.concatenate([m1, m2, m3, m4], axis=1)
    m = jnp.max(cand, axis=-1, keepdims=True)
    for _ in range(_K - 1):
        m = jnp.max(jnp.where(cand < m, cand, -jnp.inf), axis=-1,
                    keepdims=True)
    cnt = jnp.sum((s >= m).astype(jnp.int32), axis=-1, keepdims=True)

    def _exact_threshold():
        mm = jnp.max(s, axis=-1, keepdims=True)
        for _ in range(_K - 1):
            mm = jnp.max(jnp.where(s < mm, s, -jnp.inf), axis=-1,
                         keepdims=True)
        return mm

    m = jax.lax.cond(jnp.any(cnt != _K), _exact_threshold, lambda: m)
    rows = jax.lax.broadcasted_iota(jnp.int32, (_BLK, _N), 0) + i * _BLK
    cols = jax.lax.broadcasted_iota(jnp.int32, (_BLK, _N), 1)
    keep = (s >= m) & (rows != cols)
    out_ref[...] = jnp.where(keep, s, 0.0)


def kernel(feat, W, b):
    z = pl.pallas_call(
        _proj_kernel,
        grid=(_N // _BLK,),
        in_specs=[pl.BlockSpec((_BLK, _D), lambda i: (i, 0)),
                  pl.BlockSpec((_D, _H), lambda i: (0, 0)),
                  pl.BlockSpec((1, _H), lambda i: (0, 0))],
        out_specs=pl.BlockSpec((_BLK, _H), lambda i: (i, 0)),
        out_shape=jax.ShapeDtypeStruct((_N, _H), jnp.float32),
        compiler_params=_PAR,
    )(feat, W, b.reshape(1, _H))
    out = pl.pallas_call(
        _sim_kernel,
        grid=(_N // _BLK,),
        in_specs=[pl.BlockSpec((_BLK, _H), lambda i: (i, 0)),
                  pl.BlockSpec((_N, _H), lambda i: (0, 0))],
        out_specs=pl.BlockSpec((_BLK, _N), lambda i: (i, 0)),
        out_shape=jax.ShapeDtypeStruct((_N, _N), jnp.float32),
        compiler_params=_PAR,
    )(z, z)
    return out


# speedup vs baseline: 2.0864x; 1.2440x over previous
import jax
import jax.numpy as jnp
from jax.experimental import pallas as pl
from jax.experimental.pallas import tpu as pltpu

_PAR = pltpu.CompilerParams(dimension_semantics=("parallel",))

_N = 4096
_D = 512
_H = 256
_K = 15
_BLK = 512


def _proj_kernel(feat_ref, w_ref, b_ref, z_ref):
    z = jnp.dot(feat_ref[...], w_ref[...],
                preferred_element_type=jnp.float32) + b_ref[...]
    norm = jnp.sqrt(jnp.sum(z * z, axis=-1, keepdims=True))
    z_ref[...] = z / jnp.maximum(norm, 1e-12)


def _sim_kernel(zb_ref, z_ref, out_ref):
    i = pl.program_id(0)
    s = jax.lax.dot_general(zb_ref[...], z_ref[...], (((1,), (1,)), ((), ())),
                            preferred_element_type=jnp.float32)
    neg = jnp.full((_BLK, 128), -jnp.inf, dtype=jnp.float32)
    m1, m2, m3, m4 = neg, neg, neg, neg
    for c in range(_N // 128):
        v = s[:, c * 128:(c + 1) * 128]
        r = jnp.minimum(m1, v)
        m1 = jnp.maximum(m1, v)
        r, m2 = jnp.minimum(m2, r), jnp.maximum(m2, r)
        r, m3 = jnp.minimum(m3, r), jnp.maximum(m3, r)
        m4 = jnp.maximum(m4, r)
    cand = jnp.concatenate([m1, m2, m3, m4], axis=1)
    m = jnp.max(cand, axis=-1, keepdims=True)
    for _ in range(_K - 1):
        m = jnp.max(jnp.where(cand < m, cand, -jnp.inf), axis=-1,
                    keepdims=True)
    rows = jax.lax.broadcasted_iota(jnp.int32, (_BLK, _N), 0) + i * _BLK
    cols = jax.lax.broadcasted_iota(jnp.int32, (_BLK, _N), 1)
    notdiag = rows != cols
    ge = s >= m
    out_ref[...] = jnp.where(ge & notdiag, s, 0.0)
    cnt = jnp.sum(ge.astype(jnp.int32), axis=-1, keepdims=True)

    @pl.when(jnp.any(cnt != _K))
    def _exact_rewrite():
        mm = jnp.max(s, axis=-1, keepdims=True)
        for _ in range(_K - 1):
            mm = jnp.max(jnp.where(s < mm, s, -jnp.inf), axis=-1,
                         keepdims=True)
        out_ref[...] = jnp.where((s >= mm) & notdiag, s, 0.0)


def kernel(feat, W, b):
    z = pl.pallas_call(
        _proj_kernel,
        grid=(_N // _BLK,),
        in_specs=[pl.BlockSpec((_BLK, _D), lambda i: (i, 0)),
                  pl.BlockSpec((_D, _H), lambda i: (0, 0)),
                  pl.BlockSpec((1, _H), lambda i: (0, 0))],
        out_specs=pl.BlockSpec((_BLK, _H), lambda i: (i, 0)),
        out_shape=jax.ShapeDtypeStruct((_N, _H), jnp.float32),
        compiler_params=_PAR,
    )(feat, W, b.reshape(1, _H))
    out = pl.pallas_call(
        _sim_kernel,
        grid=(_N // _BLK,),
        in_specs=[pl.BlockSpec((_BLK, _H), lambda i: (i, 0)),
                  pl.BlockSpec((_N, _H), lambda i: (0, 0))],
        out_specs=pl.BlockSpec((_BLK, _N), lambda i: (i, 0)),
        out_shape=jax.ShapeDtypeStruct((_N, _N), jnp.float32),
        compiler_params=_PAR,
    )(z, z)
    return out


# speedup vs baseline: 2.4543x; 1.1763x over previous
import jax
import jax.numpy as jnp
from jax.experimental import pallas as pl
from jax.experimental.pallas import tpu as pltpu

_N = 4096
_D = 512
_H = 256
_K = 15
_BLK = 512


def _fused_kernel(feat_ref, w_ref, b_ref, out_ref, z_ref):
    i = pl.program_id(0)

    @pl.when(i == 0)
    def _project():
        z = jnp.dot(feat_ref[...], w_ref[...],
                    preferred_element_type=jnp.float32) + b_ref[...]
        norm = jnp.sqrt(jnp.sum(z * z, axis=-1, keepdims=True))
        z_ref[...] = z / jnp.maximum(norm, 1e-12)

    zb = z_ref[pl.ds(i * _BLK, _BLK), :]
    s = jax.lax.dot_general(zb, z_ref[...], (((1,), (1,)), ((), ())),
                            preferred_element_type=jnp.float32)

    neg = jnp.full((_BLK, 128), -jnp.inf, dtype=jnp.float32)
    m1, m2, m3, m4 = neg, neg, neg, neg
    for c in range(_N // 128):
        v = s[:, c * 128:(c + 1) * 128]
        r = jnp.minimum(m1, v)
        m1 = jnp.maximum(m1, v)
        r, m2 = jnp.minimum(m2, r), jnp.maximum(m2, r)
        r, m3 = jnp.minimum(m3, r), jnp.maximum(m3, r)
        m4 = jnp.maximum(m4, r)
    cand = jnp.concatenate([m1, m2, m3, m4], axis=1)
    m = jnp.max(cand, axis=-1, keepdims=True)
    for _ in range(_K - 1):
        m = jnp.max(jnp.where(cand < m, cand, -jnp.inf), axis=-1,
                    keepdims=True)

    eye = (jax.lax.broadcasted_iota(jnp.int32, (_BLK, _BLK), 0) ==
           jax.lax.broadcasted_iota(jnp.int32, (_BLK, _BLK), 1))

    def _store_masked(thr):
        out_ref[...] = jnp.where(s >= thr, s, 0.0)
        db = out_ref[:, pl.ds(i * _BLK, _BLK)]
        out_ref[:, pl.ds(i * _BLK, _BLK)] = jnp.where(eye, 0.0, db)

    ge = s >= m
    _store_masked(m)
    cnt = jnp.sum(ge.astype(jnp.int32), axis=-1, keepdims=True)

    @pl.when(jnp.any(cnt != _K))
    def _exact_rewrite():
        mm = jnp.max(s, axis=-1, keepdims=True)
        for _ in range(_K - 1):
            mm = jnp.max(jnp.where(s < mm, s, -jnp.inf), axis=-1,
                         keepdims=True)
        _store_masked(mm)


def kernel(feat, W, b):
    return pl.pallas_call(
        _fused_kernel,
        grid=(_N // _BLK,),
        in_specs=[pl.BlockSpec((_N, _D), lambda i: (0, 0)),
                  pl.BlockSpec((_D, _H), lambda i: (0, 0)),
                  pl.BlockSpec((1, _H), lambda i: (0, 0))],
        out_specs=pl.BlockSpec((_BLK, _N), lambda i: (i, 0)),
        out_shape=jax.ShapeDtypeStruct((_N, _N), jnp.float32),
        scratch_shapes=[pltpu.VMEM((_N, _H), jnp.float32)],
    )(feat, W, b.reshape(1, _H))


# speedup vs baseline: 2.8365x; 1.1558x over previous
import jax
import jax.numpy as jnp
from jax.experimental import pallas as pl
from jax.experimental.pallas import tpu as pltpu

_N = 4096
_D = 512
_H = 256
_K = 15
_BLK = 512


def _fused_kernel(feat_ref, w_ref, b_ref, out_ref, z_ref):
    i = pl.program_id(0)

    @pl.when(i == 0)
    def _project():
        z = jnp.dot(feat_ref[...], w_ref[...],
                    preferred_element_type=jnp.float32) + b_ref[...]
        norm = jnp.sqrt(jnp.sum(z * z, axis=-1, keepdims=True))
        z_ref[...] = z / jnp.maximum(norm, 1e-12)

    zb = z_ref[pl.ds(i * _BLK, _BLK), :]
    s = jax.lax.dot_general(zb, z_ref[...], (((1,), (1,)), ((), ())),
                            preferred_element_type=jnp.float32)

    neg = jnp.full((_BLK, 128), -jnp.inf, dtype=jnp.float32)
    m1, m2, m3, m4 = neg, neg, neg, neg
    for c in range(_N // 128):
        v = s[:, c * 128:(c + 1) * 128]
        r = jnp.minimum(m1, v)
        m1 = jnp.maximum(m1, v)
        r, m2 = jnp.minimum(m2, r), jnp.maximum(m2, r)
        r, m3 = jnp.minimum(m3, r), jnp.maximum(m3, r)
        m4 = jnp.maximum(m4, r)
    h1, h2, h3, h4 = m1, m2, m3, m4
    m = jnp.max(h1, axis=-1, keepdims=True)
    for _ in range(_K - 1):
        sel = h1 == m
        h1 = jnp.where(sel, h2, h1)
        h2 = jnp.where(sel, h3, h2)
        h3 = jnp.where(sel, h4, h3)
        h4 = jnp.where(sel, -jnp.inf, h4)
        m = jnp.max(h1, axis=-1, keepdims=True)

    eye = (jax.lax.broadcasted_iota(jnp.int32, (_BLK, _BLK), 0) ==
           jax.lax.broadcasted_iota(jnp.int32, (_BLK, _BLK), 1))

    def _store_masked(thr):
        out_ref[...] = jnp.where(s >= thr, s, 0.0)
        db = out_ref[:, pl.ds(i * _BLK, _BLK)]
        out_ref[:, pl.ds(i * _BLK, _BLK)] = jnp.where(eye, 0.0, db)

    _store_masked(m)

    def _count_mismatch():
        cnt = jnp.sum((s >= m).astype(jnp.int32), axis=-1, keepdims=True)
        return jnp.any(cnt != _K).astype(jnp.int32)

    bad = jax.lax.cond(jnp.any(m4 >= m), _count_mismatch,
                       lambda: jnp.zeros((), jnp.int32))

    @pl.when(bad != 0)
    def _exact_rewrite():
        mm = jnp.max(s, axis=-1, keepdims=True)
        for _ in range(_K - 1):
            mm = jnp.max(jnp.where(s < mm, s, -jnp.inf), axis=-1,
                         keepdims=True)
        _store_masked(mm)


def kernel(feat, W, b):
    return pl.pallas_call(
        _fused_kernel,
        grid=(_N // _BLK,),
        in_specs=[pl.BlockSpec((_N, _D), lambda i: (0, 0)),
                  pl.BlockSpec((_D, _H), lambda i: (0, 0)),
                  pl.BlockSpec((1, _H), lambda i: (0, 0))],
        out_specs=pl.BlockSpec((_BLK, _N), lambda i: (i, 0)),
        out_shape=jax.ShapeDtypeStruct((_N, _N), jnp.float32),
        scratch_shapes=[pltpu.VMEM((_N, _H), jnp.float32)],
    )(feat, W, b.reshape(1, _H))
